# baseline (speedup 1.0000x reference)
import functools

import jax
import jax.numpy as jnp
from jax import lax
from jax.experimental import pallas as pl
from jax.experimental.pallas import tpu as pltpu
from jax.experimental.pallas import tpu_sc as plsc

DECAY = 0.3
THRESHOLD = 0.9
CH = 40
GR = 8
NW = 32


def _labels_body(y_ref, first_ref, sel_ref, cnt_ref, cnt_acc, *, n_steps,
                 n_class):
    i = pl.program_id(0)

    @pl.when(i == 0)
    def _init():
        cnt_acc[...] = jnp.zeros_like(cnt_acc)

    y = y_ref[...]
    bn = y.shape[0]
    m = jnp.max(y, axis=1, keepdims=True)
    iota = lax.broadcasted_iota(jnp.int32, (bn, n_class), 1)
    eq = y >= m
    first = jnp.min(jnp.where(eq, iota, 2 * n_class), axis=1, keepdims=True)
    first_ref[...] = first.astype(jnp.float32)
    sel_ref[...] = jnp.where(m > THRESHOLD, m, 0.0)
    onehot = iota == first
    cnt_acc[...] += jnp.sum(onehot.astype(jnp.float32), axis=0, keepdims=True)

    @pl.when(i == n_steps - 1)
    def _fin():
        cnt_ref[...] = cnt_acc[...]


def _labels(y, bn=3200):
    n, n_class = y.shape
    n_steps = n // bn
    return pl.pallas_call(
        functools.partial(_labels_body, n_steps=n_steps, n_class=n_class),
        grid=(n_steps,),
        in_specs=[pl.BlockSpec((bn, n_class), lambda i: (i, 0))],
        out_specs=[
            pl.BlockSpec((bn, 1), lambda i: (i, 0)),
            pl.BlockSpec((bn, 1), lambda i: (i, 0)),
            pl.BlockSpec((1, n_class), lambda i: (0, 0)),
        ],
        out_shape=[
            jax.ShapeDtypeStruct((n, 1), jnp.float32),
            jax.ShapeDtypeStruct((n, 1), jnp.float32),
            jax.ShapeDtypeStruct((1, n_class), jnp.float32),
        ],
        scratch_shapes=[pltpu.VMEM((1, n_class), jnp.float32)],
        compiler_params=pltpu.CompilerParams(
            dimension_semantics=("arbitrary",)),
    )(y)


def _make_scatter(n, d, n_class):
    rows_w = n // NW
    n_chunks = rows_w // CH
    mesh = plsc.VectorSubcoreMesh(core_axis_name="c", subcore_axis_name="s")

    @functools.partial(
        pl.kernel, mesh=mesh,
        out_type=jax.ShapeDtypeStruct((NW, n_class, d), jnp.float32),
        scratch_types=[
            pltpu.VMEM((2, CH, d), jnp.float32),
            pltpu.VMEM((rows_w + 16,), jnp.int32),
            pltpu.VMEM((rows_w + 16,), jnp.float32),
            pltpu.VMEM((n_class, d), jnp.float32),
            pltpu.SemaphoreType.DMA,
            pltpu.SemaphoreType.DMA,
        ],
    )
    def scatter(feat_hbm, lab_hbm, sel_hbm, part_hbm,
                feat_v, lab_v, sel_v, accum, sem0, sem1):
        cid = lax.axis_index("c")
        sid = lax.axis_index("s")
        wid = sid * 2 + cid
        base = wid * rows_w
        sems = (sem0, sem1)

        pltpu.sync_copy(lab_hbm.at[pl.ds(base, rows_w)],
                        lab_v.at[pl.ds(0, rows_w)])
        pltpu.sync_copy(sel_hbm.at[pl.ds(base, rows_w)],
                        sel_v.at[pl.ds(0, rows_w)])

        def zrow(i, _):
            for k in range(d // 16):
                accum[i, pl.ds(k * 16, 16)] = jnp.zeros((16,), jnp.float32)
            return 0
        lax.fori_loop(0, n_class, zrow, 0)

        for b in range(2):
            pltpu.async_copy(feat_hbm.at[pl.ds(base + b * CH, CH)],
                             feat_v.at[b], sems[b])

        def compute_chunk(j, buf):
            def group(g, _):
                r0 = g * GR
                labs = lab_v[pl.ds(j * CH + r0, 16)]
                sels = sel_v[pl.ds(j * CH + r0, 16)]
                lab_l = [labs[l] for l in range(GR)]
                sel_l = [sels[l] for l in range(GR)]
                for l in range(GR):
                    lab = lab_l[l]
                    w = sel_l[l]
                    vals = [feat_v[buf, r0 + l, pl.ds(k * 16, 16)]
                            for k in range(d // 16)]
                    prods = [v * w for v in vals]
                    for k in range(d // 16):
                        plsc.addupdate(accum.at[lab, pl.ds(k * 16, 16)],
                                       prods[k])
                return 0
            lax.fori_loop(0, CH // GR, group, 0)

        def outer(jp, _):
            for b in range(2):
                j = jp * 2 + b

                @pl.when(j < n_chunks)
                def _do():
                    pltpu.make_async_copy(
                        feat_hbm.at[pl.ds(base + j * CH, CH)],
                        feat_v.at[b], sems[b]).wait()
                    compute_chunk(j, b)

                    @pl.when(j + 2 < n_chunks)
                    def _next():
                        pltpu.async_copy(
                            feat_hbm.at[pl.ds(base + (j + 2) * CH, CH)],
                            feat_v.at[b], sems[b])
            return 0

        lax.fori_loop(0, (n_chunks + 1) // 2, outer, 0)
        pltpu.sync_copy(accum, part_hbm.at[wid])

    return scatter


def _combine_body(sp_ref, tp_ref, scnt_ref, tcnt_ref, sc_ref, tc_ref,
                  loss_ref):
    nw = sp_ref.shape[0]
    s_sum = sp_ref[0]
    t_sum = tp_ref[0]
    for w in range(1, nw):
        s_sum = s_sum + sp_ref[w]
        t_sum = t_sum + tp_ref[w]
    s_n = jnp.maximum(scnt_ref[...], 1.0)
    t_n = jnp.maximum(tcnt_ref[...], 1.0)
    cur_s = s_sum / s_n
    cur_t = t_sum / t_n
    s_c = (1.0 - DECAY) * sc_ref[...] + DECAY * cur_s
    t_c = (1.0 - DECAY) * tc_ref[...] + DECAY * cur_t
    sq = (s_c - t_c) ** 2
    total = jnp.sum(jnp.sum(sq, axis=1, keepdims=True), axis=0, keepdims=True)
    loss_ref[...] = total / float(sq.shape[0] * sq.shape[1])


def kernel(s_feature, t_feature, y_s, y_t, s_centroid, t_centroid):
    n, d = s_feature.shape
    n_class = y_s.shape[1]

    s_first, s_sel, s_cnt = _labels(y_s)
    t_first, t_sel, t_cnt = _labels(y_t)

    s_lab = s_first.reshape(n).astype(jnp.int32)
    t_lab = t_first.reshape(n).astype(jnp.int32)
    s_sel2 = s_sel.reshape(n)
    t_sel2 = t_sel.reshape(n)

    scatter = _make_scatter(n, d, n_class)
    s_part = scatter(s_feature, s_lab, s_sel2)
    t_part = scatter(t_feature, t_lab, t_sel2)

    loss = pl.pallas_call(
        _combine_body,
        out_shape=jax.ShapeDtypeStruct((1, 1), jnp.float32),
    )(s_part, t_part, s_cnt.reshape(n_class, 1), t_cnt.reshape(n_class, 1),
      s_centroid, t_centroid)
    return loss[0, 0]


# speedup vs baseline: 2.0015x; 2.0015x over previous
import functools

import jax
import jax.numpy as jnp
from jax.experimental import pallas as pl
from jax.experimental.pallas import tpu as pltpu

DECAY = 0.3
THRESHOLD = 0.9


def _body(sct_ref, tct_ref, ys_ref, yt_ref, sf_ref, tf_ref, loss_ref,
          s_sum, t_sum, s_cnt, t_cnt, *, n_steps, n_class):
    i = pl.program_id(0)

    @pl.when(i == 0)
    def _init():
        s_sum[...] = jnp.zeros_like(s_sum)
        t_sum[...] = jnp.zeros_like(t_sum)
        s_cnt[...] = jnp.zeros_like(s_cnt)
        t_cnt[...] = jnp.zeros_like(t_cnt)

    def accumulate(y_ref, f_ref, sum_ref, cnt_ref):
        y = y_ref[...]
        bn = y.shape[0]
        m = jnp.max(y, axis=1, keepdims=True)
        iota = jax.lax.broadcasted_iota(jnp.int32, (bn, n_class), 1)
        eq = y >= m
        first = jnp.min(jnp.where(eq, iota, n_class), axis=1, keepdims=True)
        onehot = iota == first
        sel = jnp.where(m > THRESHOLD, m, 0.0)
        w = jnp.where(onehot, sel, 0.0)
        cnt_ref[...] += jnp.sum(onehot.astype(jnp.float32), axis=0,
                                keepdims=True)
        sum_ref[...] += jax.lax.dot_general(
            f_ref[...].astype(jnp.bfloat16), w.astype(jnp.bfloat16),
            (((0,), (0,)), ((), ())),
            preferred_element_type=jnp.float32)

    accumulate(ys_ref, sf_ref, s_sum, s_cnt)
    accumulate(yt_ref, tf_ref, t_sum, t_cnt)

    @pl.when(i == n_steps - 1)
    def _finish():
        s_n = jnp.maximum(s_cnt[...], 1.0)
        t_n = jnp.maximum(t_cnt[...], 1.0)
        cur_s = s_sum[...] / s_n
        cur_t = t_sum[...] / t_n
        s_c = (1.0 - DECAY) * sct_ref[...] + DECAY * cur_s
        t_c = (1.0 - DECAY) * tct_ref[...] + DECAY * cur_t
        sq = (s_c - t_c) ** 2
        total = jnp.sum(jnp.sum(sq, axis=1, keepdims=True), axis=0,
                        keepdims=True)
        loss_ref[...] = total / float(sq.shape[0] * sq.shape[1])


@jax.jit
def kernel(s_feature, t_feature, y_s, y_t, s_centroid, t_centroid):
    n, d = s_feature.shape
    n_class = y_s.shape[1]
    bn = 3200 if n % 3200 == 0 else n
    n_steps = n // bn

    row_spec = pl.BlockSpec((bn, n_class), lambda i: (i, 0))
    full_spec = pl.BlockSpec((d, n_class), lambda i: (0, 0))

    loss = pl.pallas_call(
        functools.partial(_body, n_steps=n_steps, n_class=n_class),
        grid=(n_steps,),
        in_specs=[full_spec, full_spec, row_spec, row_spec, row_spec, row_spec],
        out_specs=pl.BlockSpec((1, 1), lambda i: (0, 0)),
        out_shape=jax.ShapeDtypeStruct((1, 1), jnp.float32),
        scratch_shapes=[
            pltpu.VMEM((d, n_class), jnp.float32),
            pltpu.VMEM((d, n_class), jnp.float32),
            pltpu.VMEM((1, n_class), jnp.float32),
            pltpu.VMEM((1, n_class), jnp.float32),
        ],
        compiler_params=pltpu.CompilerParams(
            dimension_semantics=("arbitrary",)),
    )(s_centroid.T, t_centroid.T, y_s, y_t, s_feature, t_feature)
    return loss[0, 0]
